# Initial kernel scaffold; baseline (speedup 1.0000x reference)
import jax
import jax.numpy as jnp
from jax.experimental import pallas as pl


def kernel(graph_emb, routing_features, gate_W, expert_W, expert_b):
    raise NotImplementedError("write your pallas kernel here")


# speedup vs baseline: 3.9377x; 3.9377x over previous
import functools

import jax
import jax.numpy as jnp
from jax.experimental import pallas as pl
from jax.experimental.pallas import tpu as pltpu

B = 128
S = 16
D_IN = 2048
D_OUT = 4096
E = 8
K = 2
R_DIM = 2432

G = 8
T = (B * K) // G + (E - 1)
SLOTS = T * G
SLOT_PAD = 320
TE_PAD = 64
NT = 1024
NO = D_OUT // NT


def _router_kernel(rf_ref, gw_ref, aux_ref, se_ref, sb_ref, sw_ref):
    rf = rf_ref[...]
    gw = gw_ref[...]
    logits = jax.lax.dot_general(
        rf, gw, (((1,), (1,)), ((), ())),
        precision=jax.lax.Precision.HIGHEST,
        preferred_element_type=jnp.float32)

    eidx = jax.lax.broadcasted_iota(jnp.int32, (B, E), 1)

    l0 = jnp.max(logits, axis=1, keepdims=True)
    a0 = jnp.min(jnp.where(logits == l0, eidx, E), axis=1, keepdims=True)
    oh0 = (eidx == a0)
    masked = jnp.where(oh0, -3e38, logits)
    l1 = jnp.max(masked, axis=1, keepdims=True)
    a1 = jnp.min(jnp.where(masked == l1, eidx, E), axis=1, keepdims=True)
    oh1 = (eidx == a1)
    oh0f = oh0.astype(jnp.float32)
    oh1f = oh1.astype(jnp.float32)

    w1 = 1.0 / (1.0 + jnp.exp(l0 - l1))
    w0 = 1.0 - w1

    ex = jnp.exp(logits - l0)
    probs = ex / jnp.sum(ex, axis=1, keepdims=True)
    pmean = jnp.sum(probs, axis=0, keepdims=True) * (1.0 / B)
    cnt0 = jnp.sum(oh0f, axis=0, keepdims=True)
    cnt1 = jnp.sum(oh1f, axis=0, keepdims=True)
    cnt = cnt0 + cnt1
    frac = cnt * (1.0 / (B * K))
    aux_ref[...] = E * jnp.sum(frac * pmean, axis=1, keepdims=True)

    ri = jax.lax.broadcasted_iota(jnp.int32, (B, B), 0)
    ci = jax.lax.broadcasted_iota(jnp.int32, (B, B), 1)
    tri = (ci < ri).astype(jnp.float32)
    pc0 = jax.lax.dot_general(tri, oh0f, (((1,), (0,)), ((), ())),
                              preferred_element_type=jnp.float32)
    pc1 = jax.lax.dot_general(tri, oh1f, (((1,), (0,)), ((), ())),
                              preferred_element_type=jnp.float32)
    rank0 = jnp.sum(pc0 * oh0f, axis=1, keepdims=True)
    rank1 = (jnp.sum(pc1 * oh1f, axis=1, keepdims=True)
             + jnp.sum(cnt0 * oh1f, axis=1, keepdims=True))

    ntiles = jnp.floor((cnt + (G - 1)) * (1.0 / G))
    ei = jax.lax.broadcasted_iota(jnp.int32, (E, E), 0)
    ej = jax.lax.broadcasted_iota(jnp.int32, (E, E), 1)
    excl = (ei < ej).astype(jnp.float32)
    tbase = jax.lax.dot_general(ntiles, excl, (((1,), (0,)), ((), ())),
                                preferred_element_type=jnp.float32)
    sbase = tbase * G

    slot0 = jnp.sum(sbase * oh0f, axis=1, keepdims=True) + rank0
    slot1 = jnp.sum(sbase * oh1f, axis=1, keepdims=True) + rank1

    sio = jax.lax.broadcasted_iota(jnp.int32, (B, SLOT_PAD), 1).astype(jnp.float32)
    bvec = jax.lax.broadcasted_iota(jnp.int32, (B, 1), 0).astype(jnp.float32)
    m0 = (slot0 == sio).astype(jnp.float32)
    m1 = (slot1 == sio).astype(jnp.float32)
    sb = (jnp.sum(m0 * bvec, axis=0, keepdims=True)
          + jnp.sum(m1 * bvec, axis=0, keepdims=True))
    sw = (jnp.sum(m0 * w0, axis=0, keepdims=True)
          + jnp.sum(m1 * w1, axis=0, keepdims=True))
    sb_ref[...] = sb.astype(jnp.int32)
    sw_ref[...] = sw

    tio = jax.lax.broadcasted_iota(jnp.int32, (TE_PAD, E), 0).astype(jnp.float32)
    owned = (tio >= tbase).astype(jnp.float32)
    se_ref[...] = (jnp.sum(owned, axis=1, keepdims=True) - 1.0).astype(jnp.int32)


def _moe_kernel(se_sm, sb_sm, sw_sm, x_ref, wt_ref, bias_ref, y_ref):
    i = pl.program_id(0)
    t = pl.program_id(1)

    @pl.when(t == 0)
    def _init():
        y_ref[...] = jnp.zeros_like(y_ref)

    wsum = sw_sm[t * G]
    for j in range(1, G):
        wsum = wsum + sw_sm[t * G + j]

    @pl.when(wsum > 0.0)
    def _compute():
        xs = [x_ref[pl.ds(sb_sm[t * G + j] * S, S), :] for j in range(G)]
        xg = jnp.concatenate(xs, axis=0)
        w2 = wt_ref[0]
        acc = jax.lax.dot_general(
            xg, w2, (((1,), (1,)), ((), ())),
            preferred_element_type=jnp.float32)
        et = se_sm[t]
        acc = acc + bias_ref[pl.ds(et, 1), pl.ds(i * NT, NT)]
        for j in range(G):
            bid = sb_sm[t * G + j]
            y_ref[pl.ds(bid * S, S), :] += sw_sm[t * G + j] * acc[j * S:(j + 1) * S, :]


@functools.partial(jax.jit)
def kernel(graph_emb, routing_features, gate_W, expert_W, expert_b):
    aux, se, sb, sw = pl.pallas_call(
        _router_kernel,
        out_shape=(
            jax.ShapeDtypeStruct((1, 1), jnp.float32),
            jax.ShapeDtypeStruct((TE_PAD, 1), jnp.int32),
            jax.ShapeDtypeStruct((1, SLOT_PAD), jnp.int32),
            jax.ShapeDtypeStruct((1, SLOT_PAD), jnp.float32),
        ),
    )(routing_features, gate_W)

    se_arr = se[:T, 0]
    sb_arr = sb[0, :SLOTS]
    sw_arr = sw[0, :SLOTS]
    x = graph_emb.reshape(B * S, D_IN)

    y = pl.pallas_call(
        _moe_kernel,
        grid_spec=pltpu.PrefetchScalarGridSpec(
            num_scalar_prefetch=3,
            grid=(NO, T),
            in_specs=[
                pl.BlockSpec((B * S, D_IN), lambda i, t, *_: (0, 0)),
                pl.BlockSpec((1, NT, D_IN), lambda i, t, se, sb, sw: (se[t], i, 0)),
                pl.BlockSpec((E, D_OUT), lambda i, t, *_: (0, 0)),
            ],
            out_specs=pl.BlockSpec((B * S, NT), lambda i, t, *_: (0, i)),
        ),
        out_shape=jax.ShapeDtypeStruct((B * S, D_OUT), jnp.float32),
        compiler_params=pltpu.CompilerParams(
            dimension_semantics=("arbitrary", "arbitrary"),
        ),
    )(se_arr, sb_arr, sw_arr, x, expert_W, expert_b)

    return y.reshape(B, S, D_OUT), aux[0, 0]
